# Initial kernel scaffold; baseline (speedup 1.0000x reference)
import jax
import jax.numpy as jnp
from jax.experimental import pallas as pl


def kernel(types, pos, edge_index, batch, W1, b1, W2, b2, Wg1, bg1, Wg2, bg2, Wg3, bg3, Wo, bo):
    raise NotImplementedError("write your pallas kernel here")


# speedup vs baseline: 52.3436x; 52.3436x over previous
import functools

import jax
import jax.numpy as jnp
from jax import lax
from jax.experimental import pallas as pl
from jax.experimental.pallas import tpu as pltpu
from jax.experimental.pallas import tpu_sc as plsc

N = 100000
E = 3200000
G = 1024
H = 16

NC = 2
NS = 16
NW = NC * NS
CHUNK = 128
KB = 4
GRP = KB * CHUNK
TG = (E // NW + GRP - 1) // GRP
EPW = (TG + 2) * GRP
EPR = E // NW
NPAD = 102400
RPT = NPAD // NS
ZROWS = 128
GPAD = 1152
GRPT = GPAD // NS
NCHV = NPAD // NW // CHUNK
KBD = 8
GRPD = KBD * CHUNK
TGD = (E // NW + GRPD - 1) // GRPD

_mesh = plsc.VectorSubcoreMesh(core_axis_name="c", subcore_axis_name="s")
_f32 = jnp.float32
_params = pltpu.CompilerParams(use_tc_tiling_on_sc=False)


def _fill(ref, val, n):
    def body(i, _):
        ref[pl.ds(i * 16, 16)] = jnp.full((16,), val, _f32)
        return 0
    lax.fori_loop(0, n // 16, body, 0)


def _fill2d(ref, val, rows):
    def body(i, _):
        ref[i, :] = jnp.full((16,), val, _f32)
        return 0
    lax.fori_loop(0, rows, body, 0)


@functools.partial(
    pl.kernel,
    out_type=jax.ShapeDtypeStruct((NC, NPAD), _f32),
    mesh=_mesh,
    compiler_params=_params,
    scratch_types=(
        [pltpu.VMEM_SHARED((NPAD,), _f32),
         pltpu.VMEM((CHUNK,), _f32),
         pltpu.VMEM((ZROWS,), _f32)]
        + [pltpu.VMEM((CHUNK,), jnp.int32) for _ in range(2 * KBD)]
        + [pltpu.SemaphoreType.DMA, pltpu.SemaphoreType.DMA]
    ),
)
def _kdeg(dst_hbm, out_hbm, acc, ones_v, zer_v, *rest):
    idx = (rest[:KBD], rest[KBD:2 * KBD])
    semi, sems = rest[2 * KBD], rest[2 * KBD + 1]
    c = lax.axis_index("c")
    s = lax.axis_index("s")
    wid = c * NS + s
    _fill(ones_v, 1.0, CHUNK)
    _fill(zer_v, 0.0, ZROWS)
    base_r = s * RPT

    def zcp(t, _):
        pltpu.sync_copy(zer_v, acc.at[pl.ds(base_r + t * ZROWS, ZROWS)])
        return 0
    lax.fori_loop(0, RPT // ZROWS, zcp, 0)
    plsc.subcore_barrier()

    eb0 = wid * EPW

    def load_idx(g, p):
        cps = [pltpu.async_copy(
            dst_hbm.at[pl.ds(eb0 + g * GRPD + k * CHUNK, CHUNK)],
            idx[p][k], semi) for k in range(KBD)]
        for cp in cps:
            cp.wait()

    load_idx(0, 0)

    def half(g, p):
        cps = [pltpu.async_copy(ones_v, acc.at[idx[p][k]], sems, add=True)
               for k in range(KBD)]
        load_idx(g + 1, 1 - p)
        for cp in cps:
            cp.wait()

    def body(t, _):
        half(2 * t, 0)
        half(2 * t + 1, 1)
        return 0
    lax.fori_loop(0, TGD // 2, body, 0)
    plsc.subcore_barrier()

    pltpu.sync_copy(acc.at[pl.ds(base_r, RPT)],
                    out_hbm.at[c, pl.ds(base_r, RPT)])


@functools.partial(
    pl.kernel,
    out_type=jax.ShapeDtypeStruct((NC, NPAD, H), _f32),
    mesh=_mesh,
    compiler_params=_params,
    scratch_types=(
        [pltpu.VMEM_SHARED((NPAD, H), _f32),
         pltpu.VMEM((ZROWS, H), _f32)]
        + [pltpu.VMEM((CHUNK,), jnp.int32) for _ in range(4 * KB)]
        + [pltpu.VMEM((CHUNK, H), _f32) for _ in range(2 * KB)]
        + [pltpu.SemaphoreType.DMA, pltpu.SemaphoreType.DMA,
           pltpu.SemaphoreType.DMA]
    ),
)
def _kmsg(g_hbm, src_hbm, dst_hbm, out_hbm, acc, zer_v, *rest):
    sidx = (rest[:KB], rest[KB:2 * KB])
    didx = (rest[2 * KB:3 * KB], rest[3 * KB:4 * KB])
    rows = (rest[4 * KB:5 * KB], rest[5 * KB:6 * KB])
    semi, semg, sems = rest[6 * KB:6 * KB + 3]
    c = lax.axis_index("c")
    s = lax.axis_index("s")
    wid = c * NS + s
    _fill2d(zer_v, 0.0, ZROWS)
    base_r = s * RPT

    def zcp(t, _):
        pltpu.sync_copy(zer_v, acc.at[pl.ds(base_r + t * ZROWS, ZROWS)])
        return 0
    lax.fori_loop(0, RPT // ZROWS, zcp, 0)
    plsc.subcore_barrier()

    eb0 = wid * EPW

    def load_idx(g, p):
        cps = [pltpu.async_copy(
            src_hbm.at[pl.ds(eb0 + g * GRP + k * CHUNK, CHUNK)],
            sidx[p][k], semi) for k in range(KB)]
        cps += [pltpu.async_copy(
            dst_hbm.at[pl.ds(eb0 + g * GRP + k * CHUNK, CHUNK)],
            didx[p][k], semi) for k in range(KB)]
        for cp in cps:
            cp.wait()

    def fire_gather(p):
        for k in range(KB):
            pltpu.async_copy(g_hbm.at[sidx[p][k]], rows[p][k], semg)

    def drain_gather(p):
        for k in range(KB):
            pltpu.make_async_copy(g_hbm.at[sidx[p][k]], rows[p][k],
                                  semg).wait()

    load_idx(0, 0)
    fire_gather(0)
    load_idx(1, 1)

    def half(g, p):
        drain_gather(p)
        scps = [pltpu.async_copy(rows[p][k], acc.at[didx[p][k]], sems,
                                 add=True) for k in range(KB)]
        fire_gather(1 - p)
        for cp in scps:
            cp.wait()
        load_idx(g + 2, p)

    def body(t, _):
        half(2 * t, 0)
        half(2 * t + 1, 1)
        return 0
    lax.fori_loop(0, TG // 2, body, 0)
    drain_gather(0)
    plsc.subcore_barrier()

    pltpu.sync_copy(acc.at[pl.ds(base_r, RPT)],
                    out_hbm.at[c, pl.ds(base_r, RPT)])


@functools.partial(
    pl.kernel,
    out_type=(jax.ShapeDtypeStruct((NC, GPAD, H), _f32),
              jax.ShapeDtypeStruct((NC, GPAD), _f32)),
    mesh=_mesh,
    compiler_params=_params,
    scratch_types=(
        [pltpu.VMEM_SHARED((GPAD, H), _f32),
         pltpu.VMEM_SHARED((GPAD,), _f32),
         pltpu.VMEM((GRPT, H), _f32),
         pltpu.VMEM((80,), _f32),
         pltpu.VMEM((CHUNK,), _f32),
         pltpu.VMEM((CHUNK,), jnp.int32),
         pltpu.VMEM((CHUNK, H), _f32),
         pltpu.SemaphoreType.DMA, pltpu.SemaphoreType.DMA]
    ),
)
def _kpool(z_hbm, b_hbm, outs_hbm, outc_hbm, accs, accc, zer_v, zerc_v,
           ones_v, bidx, rows, semi, sems):
    c = lax.axis_index("c")
    s = lax.axis_index("s")
    wid = c * NS + s
    _fill2d(zer_v, 0.0, GRPT)
    _fill(zerc_v, 0.0, 80)
    _fill(ones_v, 1.0, CHUNK)
    base_r = s * GRPT
    pltpu.sync_copy(zer_v, accs.at[pl.ds(base_r, GRPT)])
    pltpu.sync_copy(zerc_v.at[pl.ds(0, GRPT)], accc.at[pl.ds(base_r, GRPT)])
    plsc.subcore_barrier()

    nb0 = wid * (NPAD // NW)

    def body(g, _):
        nb = nb0 + g * CHUNK
        cp1 = pltpu.async_copy(b_hbm.at[pl.ds(nb, CHUNK)], bidx, semi)
        cp2 = pltpu.async_copy(z_hbm.at[pl.ds(nb, CHUNK)], rows, semi)
        cp1.wait()
        cp2.wait()
        pltpu.async_copy(rows, accs.at[bidx], sems, add=True).wait()
        pltpu.async_copy(ones_v, accc.at[bidx], sems, add=True).wait()
        return 0
    lax.fori_loop(0, NCHV, body, 0)
    plsc.subcore_barrier()

    pltpu.sync_copy(accs.at[pl.ds(base_r, GRPT)],
                    outs_hbm.at[c, pl.ds(base_r, GRPT)])
    pltpu.sync_copy(accc.at[pl.ds(base_r, GRPT)],
                    outc_hbm.at[c, pl.ds(base_r, GRPT)])


def _padrows(x):
    return jnp.concatenate([x, jnp.zeros((NPAD - N, H), _f32)], axis=0)


def _slab(e):
    pad = N + (jnp.arange(EPW - EPR, dtype=jnp.int32) % 1024)
    pad2 = jnp.broadcast_to(pad, (NW, EPW - EPR))
    return jnp.concatenate([e.reshape(NW, EPR), pad2], axis=1).reshape(-1)


def kernel(types, pos, edge_index, batch, W1, b1, W2, b2,
           Wg1, bg1, Wg2, bg2, Wg3, bg3, Wo, bo):
    src_p = _slab(edge_index[0].astype(jnp.int32))
    dst_p = _slab(edge_index[1].astype(jnp.int32))

    deg2 = _kdeg(dst_p)
    deg = deg2[0, :N] + deg2[1, :N] + 1.0
    dinv = lax.rsqrt(deg)
    d2 = dinv * dinv

    x_t = jax.nn.one_hot(types, 3, dtype=_f32) @ W1 + b1
    x_p = jax.nn.one_hot(pos, 3, dtype=_f32) @ W2 + b2
    z = jnp.concatenate([x_t, x_p], axis=-1)

    def gcn_layer(h, b, relu_after):
        g = _padrows(dinv[:, None] * h)
        T2 = _kmsg(g, src_p, dst_p)
        t = T2[0, :N] + T2[1, :N]
        agg = dinv[:, None] * t + d2[:, None] * h + b
        return jax.nn.relu(agg) if relu_after else agg

    z2 = gcn_layer(z @ Wg1, bg1, True)
    z3 = gcn_layer(z2 @ Wg2, bg2, True)
    z4 = gcn_layer(z3 @ Wg3, bg3, False)

    batch_p = jnp.concatenate(
        [batch.astype(jnp.int32),
         G + (jnp.arange(NPAD - N, dtype=jnp.int32) % (GPAD - G))])
    S2, C2 = _kpool(_padrows(z4), batch_p)
    sums = S2[0, :G] + S2[1, :G]
    cnt = C2[0, :G] + C2[1, :G]
    pooled = sums / jnp.maximum(cnt, 1.0)[:, None]
    return pooled @ Wo + bo


# speedup vs baseline: 54.3816x; 1.0389x over previous
import functools

import jax
import jax.numpy as jnp
from jax import lax
from jax.experimental import pallas as pl
from jax.experimental.pallas import tpu as pltpu
from jax.experimental.pallas import tpu_sc as plsc

N = 100000
E = 3200000
G = 1024
H = 16

NC = 2
NS = 16
NW = NC * NS
CHUNK = 128
KB = 4
GRP = KB * CHUNK
EPR = E // NW
TG = (EPR + GRP - 1) // GRP
GTAIL = EPR // GRP
TVAL = EPR - GTAIL * GRP
EPW = (TG + 2) * GRP
EPAD = E + (EPW - EPR)
KBD = 8
GRPD = KBD * CHUNK
TGD = (EPR + GRPD - 1) // GRPD
GTAILD = EPR // GRPD
TVALD = EPR - GTAILD * GRPD
NPAD = 102400
RPT = NPAD // NS
ZROWS = 128
GPAD = 1152
GRPT = GPAD // NS
NCHV = NPAD // NW // CHUNK
NB8 = NPAD // 8

_mesh = plsc.VectorSubcoreMesh(core_axis_name="c", subcore_axis_name="s")
_f32 = jnp.float32
_i32 = jnp.int32
_params = pltpu.CompilerParams(use_tc_tiling_on_sc=False,
                               needs_layout_passes=False)


def _fill(ref, val, n):
    def body(i, _):
        ref[pl.ds(i * 16, 16)] = jnp.full((16,), val, _f32)
        return 0
    lax.fori_loop(0, n // 16, body, 0)


def _fill2d(ref, val, rows):
    def body(i, _):
        ref[i, :] = jnp.full((16,), val, _f32)
        return 0
    lax.fori_loop(0, rows, body, 0)


def _mask_tail(didx_bufs, s, nbuf, valid):
    trash = N + lax.iota(_i32, 16) * 60 + s * 3
    for k in range(nbuf):
        lo = valid - k * CHUNK
        for i in range(8):
            if lo < (i + 1) * 16:
                didx_bufs[k][pl.ds(i * 16, 16)] = trash


@functools.partial(
    pl.kernel,
    out_type=jax.ShapeDtypeStruct((NC, NPAD), _f32),
    mesh=_mesh,
    compiler_params=_params,
    scratch_types=(
        [pltpu.VMEM_SHARED((NPAD,), _f32),
         pltpu.VMEM((CHUNK,), _f32),
         pltpu.VMEM((ZROWS,), _f32)]
        + [pltpu.VMEM((CHUNK,), _i32) for _ in range(2 * KBD)]
        + [pltpu.SemaphoreType.DMA, pltpu.SemaphoreType.DMA]
    ),
)
def _kdeg(ei_hbm, out_hbm, acc, ones_v, zer_v, *rest):
    idx = (rest[:KBD], rest[KBD:2 * KBD])
    semi, sems = rest[2 * KBD], rest[2 * KBD + 1]
    c = lax.axis_index("c")
    s = lax.axis_index("s")
    wid = c * NS + s
    _fill(ones_v, 1.0, CHUNK)
    _fill(zer_v, 0.0, ZROWS)
    base_r = s * RPT

    def zcp(t, _):
        pltpu.sync_copy(zer_v, acc.at[pl.ds(base_r + t * ZROWS, ZROWS)])
        return 0
    lax.fori_loop(0, RPT // ZROWS, zcp, 0)
    plsc.subcore_barrier()

    eb0 = wid * EPR

    def load_idx(g, p):
        cps = [pltpu.async_copy(
            ei_hbm.at[1, pl.ds(eb0 + g * GRPD + k * CHUNK, CHUNK)],
            idx[p][k], semi) for k in range(KBD)]
        for cp in cps:
            cp.wait()

        @pl.when(g == GTAILD)
        def _():
            _mask_tail(idx[p], s, KBD, TVALD)

    load_idx(0, 0)

    def half(g, p):
        cps = [pltpu.async_copy(ones_v, acc.at[idx[p][k]], sems, add=True)
               for k in range(KBD)]
        load_idx(g + 1, 1 - p)
        for cp in cps:
            cp.wait()

    def body(t, _):
        half(2 * t, 0)
        half(2 * t + 1, 1)
        return 0
    lax.fori_loop(0, TGD // 2, body, 0)
    plsc.subcore_barrier()

    pltpu.sync_copy(acc.at[pl.ds(base_r, RPT)],
                    out_hbm.at[c, pl.ds(base_r, RPT)])


@functools.partial(
    pl.kernel,
    out_type=jax.ShapeDtypeStruct((NC, NPAD * H), _f32),
    mesh=_mesh,
    compiler_params=_params,
    scratch_types=(
        [pltpu.VMEM_SHARED((NPAD * H,), _f32),
         pltpu.VMEM((2048,), _f32)]
        + [pltpu.VMEM((CHUNK,), _i32) for _ in range(6 * KB)]
        + [pltpu.VMEM((CHUNK,), _i32) for _ in range(2 * KB)]
        + [pltpu.VMEM((CHUNK,), _f32) for _ in range(2 * KB)]
        + [pltpu.SemaphoreType.DMA, pltpu.SemaphoreType.DMA,
           pltpu.SemaphoreType.DMA]
    ),
)
def _kcls(pk_hbm, ei_hbm, out_hbm, acc, zer_v, *rest):
    sidx = (rest[:KB], rest[KB:2 * KB])
    didx = (rest[2 * KB:3 * KB], rest[3 * KB:4 * KB])
    pvb = (rest[4 * KB:5 * KB], rest[5 * KB:6 * KB])
    tgtb = (rest[6 * KB:7 * KB], rest[7 * KB:8 * KB])
    dvb = (rest[8 * KB:9 * KB], rest[9 * KB:10 * KB])
    semi, semg, sems = rest[10 * KB:10 * KB + 3]
    c = lax.axis_index("c")
    s = lax.axis_index("s")
    wid = c * NS + s
    _fill(zer_v, 0.0, 2048)
    base_w = s * RPT * H

    def zcp(t, _):
        pltpu.sync_copy(zer_v, acc.at[pl.ds(base_w + t * 2048, 2048)])
        return 0
    lax.fori_loop(0, RPT * H // 2048, zcp, 0)
    plsc.subcore_barrier()

    eb0 = wid * EPR

    def load_idx(g, p):
        cps = [pltpu.async_copy(
            ei_hbm.at[0, pl.ds(eb0 + g * GRP + k * CHUNK, CHUNK)],
            sidx[p][k], semi) for k in range(KB)]
        cps += [pltpu.async_copy(
            ei_hbm.at[1, pl.ds(eb0 + g * GRP + k * CHUNK, CHUNK)],
            didx[p][k], semi) for k in range(KB)]
        for cp in cps:
            cp.wait()

        @pl.when(g == GTAIL)
        def _():
            _mask_tail(didx[p], s, KB, TVAL)

    def fire_gather(p):
        for k in range(KB):
            pltpu.async_copy(pk_hbm.at[sidx[p][k]], pvb[p][k], semg)

    def drain_gather(p):
        for k in range(KB):
            pltpu.make_async_copy(pk_hbm.at[sidx[p][k]], pvb[p][k],
                                  semg).wait()

    def compute(p):
        for k in range(KB):
            for i in range(8):
                sl = pl.ds(i * 16, 16)
                pv = pvb[p][k][sl]
                cv = pv & 15
                dvb[p][k][sl] = plsc.bitcast(pv - cv, _f32)
                tgtb[p][k][sl] = didx[p][k][sl] * 16 + cv

    load_idx(0, 0)
    fire_gather(0)
    load_idx(1, 1)

    def half(g, p):
        drain_gather(p)
        fire_gather(1 - p)
        compute(p)
        scps = [pltpu.async_copy(dvb[p][k], acc.at[tgtb[p][k]], sems,
                                 add=True) for k in range(KB)]
        for cp in scps:
            cp.wait()
        load_idx(g + 2, p)

    def body(t, _):
        half(2 * t, 0)
        half(2 * t + 1, 1)
        return 0
    lax.fori_loop(0, TG // 2, body, 0)
    drain_gather(0)

    iota16 = lax.iota(_i32, 16)
    nb0 = wid * (NPAD // NW)

    def npass(t, _):
        nb = nb0 + t * CHUNK
        pltpu.sync_copy(pk_hbm.at[pl.ds(nb, CHUNK)], pvb[0][0])
        for i in range(8):
            sl = pl.ds(i * 16, 16)
            pv = pvb[0][0][sl]
            cv = pv & 15
            dvb[0][0][sl] = plsc.bitcast(pv - cv, _f32)
            tgtb[0][0][sl] = (nb + i * 16) * 16 + iota16 * 16 + cv
        pltpu.async_copy(dvb[0][0], acc.at[tgtb[0][0]], sems,
                         add=True).wait()
        return 0
    lax.fori_loop(0, NCHV, npass, 0)
    plsc.subcore_barrier()

    pltpu.sync_copy(acc.at[pl.ds(base_w, RPT * H)],
                    out_hbm.at[c, pl.ds(base_w, RPT * H)])


@functools.partial(
    pl.kernel,
    out_type=jax.ShapeDtypeStruct((NC, NPAD, H), _f32),
    mesh=_mesh,
    compiler_params=_params,
    scratch_types=(
        [pltpu.VMEM_SHARED((NPAD, H), _f32),
         pltpu.VMEM((ZROWS, H), _f32)]
        + [pltpu.VMEM((CHUNK,), _i32) for _ in range(4 * KB)]
        + [pltpu.VMEM((CHUNK, H), _f32) for _ in range(2 * KB)]
        + [pltpu.SemaphoreType.DMA, pltpu.SemaphoreType.DMA,
           pltpu.SemaphoreType.DMA]
    ),
)
def _kmsg(g_hbm, ei_hbm, out_hbm, acc, zer_v, *rest):
    sidx = (rest[:KB], rest[KB:2 * KB])
    didx = (rest[2 * KB:3 * KB], rest[3 * KB:4 * KB])
    rows = (rest[4 * KB:5 * KB], rest[5 * KB:6 * KB])
    semi, semg, sems = rest[6 * KB:6 * KB + 3]
    c = lax.axis_index("c")
    s = lax.axis_index("s")
    wid = c * NS + s
    _fill2d(zer_v, 0.0, ZROWS)
    base_r = s * RPT

    def zcp(t, _):
        pltpu.sync_copy(zer_v, acc.at[pl.ds(base_r + t * ZROWS, ZROWS)])
        return 0
    lax.fori_loop(0, RPT // ZROWS, zcp, 0)
    plsc.subcore_barrier()

    eb0 = wid * EPR

    def load_idx(g, p):
        cps = [pltpu.async_copy(
            ei_hbm.at[0, pl.ds(eb0 + g * GRP + k * CHUNK, CHUNK)],
            sidx[p][k], semi) for k in range(KB)]
        cps += [pltpu.async_copy(
            ei_hbm.at[1, pl.ds(eb0 + g * GRP + k * CHUNK, CHUNK)],
            didx[p][k], semi) for k in range(KB)]
        for cp in cps:
            cp.wait()

        @pl.when(g == GTAIL)
        def _():
            _mask_tail(didx[p], s, KB, TVAL)

    def fire_gather(p):
        for k in range(KB):
            pltpu.async_copy(g_hbm.at[sidx[p][k]], rows[p][k], semg)

    def drain_gather(p):
        for k in range(KB):
            pltpu.make_async_copy(g_hbm.at[sidx[p][k]], rows[p][k],
                                  semg).wait()

    load_idx(0, 0)
    fire_gather(0)
    load_idx(1, 1)

    def half(g, p):
        drain_gather(p)
        scps = [pltpu.async_copy(rows[p][k], acc.at[didx[p][k]], sems,
                                 add=True) for k in range(KB)]
        fire_gather(1 - p)
        for cp in scps:
            cp.wait()
        load_idx(g + 2, p)

    def body(t, _):
        half(2 * t, 0)
        half(2 * t + 1, 1)
        return 0
    lax.fori_loop(0, TG // 2, body, 0)
    drain_gather(0)
    plsc.subcore_barrier()

    pltpu.sync_copy(acc.at[pl.ds(base_r, RPT)],
                    out_hbm.at[c, pl.ds(base_r, RPT)])


@functools.partial(
    pl.kernel,
    out_type=(jax.ShapeDtypeStruct((NC, GPAD, H), _f32),
              jax.ShapeDtypeStruct((NC, GPAD), _f32)),
    mesh=_mesh,
    compiler_params=_params,
    scratch_types=(
        [pltpu.VMEM_SHARED((GPAD, H), _f32),
         pltpu.VMEM_SHARED((GPAD,), _f32),
         pltpu.VMEM((GRPT, H), _f32),
         pltpu.VMEM((80,), _f32),
         pltpu.VMEM((CHUNK,), _f32),
         pltpu.VMEM((CHUNK,), _i32),
         pltpu.VMEM((CHUNK, H), _f32),
         pltpu.SemaphoreType.DMA, pltpu.SemaphoreType.DMA]
    ),
)
def _kpool(z_hbm, b_hbm, outs_hbm, outc_hbm, accs, accc, zer_v, zerc_v,
           ones_v, bidx, rows, semi, sems):
    c = lax.axis_index("c")
    s = lax.axis_index("s")
    wid = c * NS + s
    _fill2d(zer_v, 0.0, GRPT)
    _fill(zerc_v, 0.0, 80)
    _fill(ones_v, 1.0, CHUNK)
    base_r = s * GRPT
    pltpu.sync_copy(zer_v, accs.at[pl.ds(base_r, GRPT)])
    pltpu.sync_copy(zerc_v.at[pl.ds(0, GRPT)], accc.at[pl.ds(base_r, GRPT)])
    plsc.subcore_barrier()

    nb0 = wid * (NPAD // NW)

    def body(g, _):
        nb = nb0 + g * CHUNK
        cp1 = pltpu.async_copy(b_hbm.at[pl.ds(nb, CHUNK)], bidx, semi)
        cp2 = pltpu.async_copy(z_hbm.at[pl.ds(nb, CHUNK)], rows, semi)
        cp1.wait()
        cp2.wait()
        pltpu.async_copy(rows, accs.at[bidx], sems, add=True).wait()
        pltpu.async_copy(ones_v, accc.at[bidx], sems, add=True).wait()
        return 0
    lax.fori_loop(0, NCHV, body, 0)
    plsc.subcore_barrier()

    pltpu.sync_copy(accs.at[pl.ds(base_r, GRPT)],
                    outs_hbm.at[c, pl.ds(base_r, GRPT)])
    pltpu.sync_copy(accc.at[pl.ds(base_r, GRPT)],
                    outc_hbm.at[c, pl.ds(base_r, GRPT)])


_HI = lax.Precision.HIGHEST


def kernel(types, pos, edge_index, batch, W1, b1, W2, b2,
           Wg1, bg1, Wg2, bg2, Wg3, bg3, Wo, bo):
    eye8 = jnp.eye(8, dtype=_f32)
    epad = jnp.concatenate(
        [edge_index.astype(_i32),
         jnp.full((2, EPAD - E), N, _i32)], axis=1)

    deg2 = _kdeg(epad)
    deg_pad = deg2[0] + deg2[1] + 1.0
    dinv_pad = lax.rsqrt(deg_pad)
    dinv_big = jnp.matmul(dinv_pad.reshape(NB8, 8),
                          jnp.kron(eye8, jnp.ones((1, H), _f32)),
                          precision=_HI)

    c_cls = types.astype(_i32) * 3 + pos.astype(_i32)
    c_pad = jnp.concatenate([c_cls, jnp.zeros((NPAD - N,), _i32)])
    pk = (lax.bitcast_convert_type(dinv_pad, _i32) & (-16)) | c_pad
    S2 = _kcls(pk, epad)
    S_big = (S2[0] + S2[1]).reshape(NB8, 128)

    tidx = jnp.arange(9) // 3
    pidx = jnp.arange(9) % 3
    ztab = jnp.concatenate(
        [jnp.matmul(jax.nn.one_hot(tidx, 3, dtype=_f32), W1) + b1[None, :],
         jnp.matmul(jax.nn.one_hot(pidx, 3, dtype=_f32), W2) + b2[None, :]],
        axis=1)
    tab16 = jnp.zeros((16, H), _f32).at[:9].set(jnp.matmul(ztab, Wg1))
    z2_big = jax.nn.relu(dinv_big * jnp.matmul(S_big,
                                               jnp.kron(eye8, tab16),
                                               precision=_HI)
                         + jnp.tile(bg1, 8))

    def layer(z_big, W, b, relu_after):
        h_big = jnp.matmul(z_big, jnp.kron(eye8, W))
        g_big = dinv_big * h_big
        T2 = _kmsg(g_big.reshape(NPAD, H), epad)
        t_big = (T2[0] + T2[1]).reshape(NB8, 128)
        agg = dinv_big * (t_big + g_big) + jnp.tile(b, 8)
        return jax.nn.relu(agg) if relu_after else agg

    z3_big = layer(z2_big, Wg2, bg2, True)
    z4_big = layer(z3_big, Wg3, bg3, False)

    batch_p = jnp.concatenate(
        [batch.astype(_i32),
         G + (jnp.arange(NPAD - N, dtype=_i32) % (GPAD - G))])
    S2p, C2p = _kpool(z4_big.reshape(NPAD, H), batch_p)
    sums = S2p[0, :G] + S2p[1, :G]
    cnt = C2p[0, :G] + C2p[1, :G]
    pooled = sums / jnp.maximum(cnt, 1.0)[:, None]
    return jnp.matmul(pooled, Wo) + bo


# speedup vs baseline: 55.3027x; 1.0169x over previous
import functools

import jax
import jax.numpy as jnp
from jax import lax
from jax.experimental import pallas as pl
from jax.experimental.pallas import tpu as pltpu
from jax.experimental.pallas import tpu_sc as plsc

N = 100000
E = 3200000
G = 1024
H = 16

NC = 2
NS = 16
NW = NC * NS
CHUNK = 128
KB = 4
GRP = KB * CHUNK
EPR = E // NW
TG = (EPR + GRP - 1) // GRP
GTAIL = EPR // GRP
TVAL = EPR - GTAIL * GRP
EPW = (TG + 2) * GRP
EPAD = E + (EPW - EPR)
KBD = 8
GRPD = KBD * CHUNK
TGD = (EPR + GRPD - 1) // GRPD
GTAILD = EPR // GRPD
TVALD = EPR - GTAILD * GRPD
NPAD = 102400
RPT = NPAD // NS
ZROWS = 128
GPAD = 1152
GRPT = GPAD // NS
NCHV = NPAD // NW // CHUNK
NB8 = NPAD // 8

_mesh = plsc.VectorSubcoreMesh(core_axis_name="c", subcore_axis_name="s")
_f32 = jnp.float32
_i32 = jnp.int32
_params = pltpu.CompilerParams(use_tc_tiling_on_sc=False,
                               needs_layout_passes=False)


def _fill(ref, val, n):
    def body(i, _):
        ref[pl.ds(i * 16, 16)] = jnp.full((16,), val, _f32)
        return 0
    lax.fori_loop(0, n // 16, body, 0)


def _fill2d(ref, val, rows):
    def body(i, _):
        ref[i, :] = jnp.full((16,), val, _f32)
        return 0
    lax.fori_loop(0, rows, body, 0)


def _mask_tail(didx_bufs, s, nbuf, valid):
    trash = N + lax.iota(_i32, 16) * 60 + s * 3
    for k in range(nbuf):
        lo = valid - k * CHUNK
        for i in range(8):
            if lo < (i + 1) * 16:
                didx_bufs[k][pl.ds(i * 16, 16)] = trash


@functools.partial(
    pl.kernel,
    out_type=jax.ShapeDtypeStruct((NC, NPAD), _f32),
    mesh=_mesh,
    compiler_params=_params,
    scratch_types=(
        [pltpu.VMEM_SHARED((NPAD,), _f32),
         pltpu.VMEM((CHUNK,), _f32),
         pltpu.VMEM((ZROWS,), _f32)]
        + [pltpu.VMEM((CHUNK,), _i32) for _ in range(2 * KBD)]
        + [pltpu.SemaphoreType.DMA, pltpu.SemaphoreType.DMA]
    ),
)
def _kdeg(ei_hbm, out_hbm, acc, ones_v, zer_v, *rest):
    idx = (rest[:KBD], rest[KBD:2 * KBD])
    semi, sems = rest[2 * KBD], rest[2 * KBD + 1]
    c = lax.axis_index("c")
    s = lax.axis_index("s")
    wid = c * NS + s
    _fill(ones_v, 1.0, CHUNK)
    _fill(zer_v, 0.0, ZROWS)
    base_r = s * RPT

    def zcp(t, _):
        pltpu.sync_copy(zer_v, acc.at[pl.ds(base_r + t * ZROWS, ZROWS)])
        return 0
    lax.fori_loop(0, RPT // ZROWS, zcp, 0)
    plsc.subcore_barrier()

    eb0 = wid * EPR

    def load_idx(g, p):
        cps = [pltpu.async_copy(
            ei_hbm.at[1, pl.ds(eb0 + g * GRPD + k * CHUNK, CHUNK)],
            idx[p][k], semi) for k in range(KBD)]
        for cp in cps:
            cp.wait()

        @pl.when(g == GTAILD)
        def _():
            _mask_tail(idx[p], s, KBD, TVALD)

    load_idx(0, 0)

    def half(g, p):
        cps = [pltpu.async_copy(ones_v, acc.at[idx[p][k]], sems, add=True)
               for k in range(KBD)]
        load_idx(g + 1, 1 - p)
        for cp in cps:
            cp.wait()

    def body(t, _):
        half(2 * t, 0)
        half(2 * t + 1, 1)
        return 0
    lax.fori_loop(0, TGD // 2, body, 0)
    plsc.subcore_barrier()

    pltpu.sync_copy(acc.at[pl.ds(base_r, RPT)],
                    out_hbm.at[c, pl.ds(base_r, RPT)])


@functools.partial(
    pl.kernel,
    out_type=jax.ShapeDtypeStruct((NC, NPAD * H), _f32),
    mesh=_mesh,
    compiler_params=_params,
    scratch_types=(
        [pltpu.VMEM_SHARED((NPAD * H,), _f32),
         pltpu.VMEM((2048,), _f32)]
        + [pltpu.VMEM((CHUNK,), _i32) for _ in range(6 * KB)]
        + [pltpu.VMEM((CHUNK,), _i32) for _ in range(2 * KB)]
        + [pltpu.VMEM((CHUNK,), _f32) for _ in range(2 * KB)]
        + [pltpu.SemaphoreType.DMA, pltpu.SemaphoreType.DMA,
           pltpu.SemaphoreType.DMA]
    ),
)
def _kcls(pk_hbm, ei_hbm, out_hbm, acc, zer_v, *rest):
    sidx = (rest[:KB], rest[KB:2 * KB])
    didx = (rest[2 * KB:3 * KB], rest[3 * KB:4 * KB])
    pvb = (rest[4 * KB:5 * KB], rest[5 * KB:6 * KB])
    tgtb = (rest[6 * KB:7 * KB], rest[7 * KB:8 * KB])
    dvb = (rest[8 * KB:9 * KB], rest[9 * KB:10 * KB])
    semi, semg, sems = rest[10 * KB:10 * KB + 3]
    c = lax.axis_index("c")
    s = lax.axis_index("s")
    wid = c * NS + s
    _fill(zer_v, 0.0, 2048)
    base_w = s * RPT * H

    def zcp(t, _):
        pltpu.sync_copy(zer_v, acc.at[pl.ds(base_w + t * 2048, 2048)])
        return 0
    lax.fori_loop(0, RPT * H // 2048, zcp, 0)
    plsc.subcore_barrier()

    eb0 = wid * EPR

    def load_idx(g, p):
        cps = [pltpu.async_copy(
            ei_hbm.at[0, pl.ds(eb0 + g * GRP + k * CHUNK, CHUNK)],
            sidx[p][k], semi) for k in range(KB)]
        cps += [pltpu.async_copy(
            ei_hbm.at[1, pl.ds(eb0 + g * GRP + k * CHUNK, CHUNK)],
            didx[p][k], semi) for k in range(KB)]
        for cp in cps:
            cp.wait()

        @pl.when(g == GTAIL)
        def _():
            _mask_tail(didx[p], s, KB, TVAL)

    def fire_gather(p):
        for k in range(KB):
            pltpu.async_copy(pk_hbm.at[sidx[p][k]], pvb[p][k], semg)

    def drain_gather(p):
        for k in range(KB):
            pltpu.make_async_copy(pk_hbm.at[sidx[p][k]], pvb[p][k],
                                  semg).wait()

    def compute(p):
        for k in range(KB):
            for i in range(8):
                sl = pl.ds(i * 16, 16)
                pv = pvb[p][k][sl]
                cv = pv & 15
                dvb[p][k][sl] = plsc.bitcast(pv - cv, _f32)
                tgtb[p][k][sl] = didx[p][k][sl] * 16 + cv

    load_idx(0, 0)
    fire_gather(0)
    load_idx(1, 1)

    def half(g, p):
        drain_gather(p)
        fire_gather(1 - p)
        compute(p)
        scps = [pltpu.async_copy(dvb[p][k], acc.at[tgtb[p][k]], sems,
                                 add=True) for k in range(KB)]
        for cp in scps:
            cp.wait()
        load_idx(g + 2, p)

    def body(t, _):
        half(2 * t, 0)
        half(2 * t + 1, 1)
        return 0
    lax.fori_loop(0, TG // 2, body, 0)
    drain_gather(0)

    iota16 = lax.iota(_i32, 16)
    nb0 = wid * (NPAD // NW)

    def npass(t, _):
        nb = nb0 + t * CHUNK
        pltpu.sync_copy(pk_hbm.at[pl.ds(nb, CHUNK)], pvb[0][0])
        for i in range(8):
            sl = pl.ds(i * 16, 16)
            pv = pvb[0][0][sl]
            cv = pv & 15
            dvb[0][0][sl] = plsc.bitcast(pv - cv, _f32)
            tgtb[0][0][sl] = (nb + i * 16) * 16 + iota16 * 16 + cv
        pltpu.async_copy(dvb[0][0], acc.at[tgtb[0][0]], sems,
                         add=True).wait()
        return 0
    lax.fori_loop(0, NCHV, npass, 0)
    plsc.subcore_barrier()

    pltpu.sync_copy(acc.at[pl.ds(base_w, RPT * H)],
                    out_hbm.at[c, pl.ds(base_w, RPT * H)])


@functools.partial(
    pl.kernel,
    out_type=jax.ShapeDtypeStruct((NC, NPAD, H), _f32),
    mesh=_mesh,
    compiler_params=_params,
    scratch_types=(
        [pltpu.VMEM_SHARED((NPAD, H), _f32),
         pltpu.VMEM((ZROWS, H), _f32)]
        + [pltpu.VMEM((CHUNK,), _i32) for _ in range(4 * KB)]
        + [pltpu.VMEM((CHUNK, H), _f32) for _ in range(2 * KB)]
        + [pltpu.SemaphoreType.DMA, pltpu.SemaphoreType.DMA,
           pltpu.SemaphoreType.DMA]
    ),
)
def _kmsg(g_hbm, ei_hbm, out_hbm, acc, zer_v, *rest):
    sidx = (rest[:KB], rest[KB:2 * KB])
    didx = (rest[2 * KB:3 * KB], rest[3 * KB:4 * KB])
    rows = (rest[4 * KB:5 * KB], rest[5 * KB:6 * KB])
    semi, semg, sems = rest[6 * KB:6 * KB + 3]
    c = lax.axis_index("c")
    s = lax.axis_index("s")
    wid = c * NS + s
    _fill2d(zer_v, 0.0, ZROWS)
    base_r = s * RPT

    def zcp(t, _):
        pltpu.sync_copy(zer_v, acc.at[pl.ds(base_r + t * ZROWS, ZROWS)])
        return 0
    lax.fori_loop(0, RPT // ZROWS, zcp, 0)
    plsc.subcore_barrier()

    eb0 = wid * EPR

    def load_idx(g, p):
        cps = [pltpu.async_copy(
            ei_hbm.at[0, pl.ds(eb0 + g * GRP + k * CHUNK, CHUNK)],
            sidx[p][k], semi) for k in range(KB)]
        cps += [pltpu.async_copy(
            ei_hbm.at[1, pl.ds(eb0 + g * GRP + k * CHUNK, CHUNK)],
            didx[p][k], semi) for k in range(KB)]
        for cp in cps:
            cp.wait()

        @pl.when(g == GTAIL)
        def _():
            _mask_tail(didx[p], s, KB, TVAL)

    def fire_gather(p):
        for k in range(KB):
            pltpu.async_copy(g_hbm.at[sidx[p][k]], rows[p][k], semg)

    def drain_gather(p):
        for k in range(KB):
            pltpu.make_async_copy(g_hbm.at[sidx[p][k]], rows[p][k],
                                  semg).wait()

    load_idx(0, 0)
    fire_gather(0)
    load_idx(1, 1)

    def half(g, p):
        drain_gather(p)
        scps = [pltpu.async_copy(rows[p][k], acc.at[didx[p][k]], sems,
                                 add=True) for k in range(KB)]
        fire_gather(1 - p)
        for cp in scps:
            cp.wait()
        load_idx(g + 2, p)

    def body(t, _):
        half(2 * t, 0)
        half(2 * t + 1, 1)
        return 0
    lax.fori_loop(0, TG // 2, body, 0)
    drain_gather(0)
    plsc.subcore_barrier()

    pltpu.sync_copy(acc.at[pl.ds(base_r, RPT)],
                    out_hbm.at[c, pl.ds(base_r, RPT)])


@functools.partial(
    pl.kernel,
    out_type=(jax.ShapeDtypeStruct((NC, GPAD, H), _f32),
              jax.ShapeDtypeStruct((NC, GPAD), _f32)),
    mesh=_mesh,
    compiler_params=_params,
    scratch_types=(
        [pltpu.VMEM_SHARED((GPAD, H), _f32),
         pltpu.VMEM_SHARED((GPAD,), _f32),
         pltpu.VMEM((GRPT, H), _f32),
         pltpu.VMEM((80,), _f32),
         pltpu.VMEM((CHUNK,), _f32),
         pltpu.VMEM((CHUNK,), _i32),
         pltpu.VMEM((CHUNK, H), _f32),
         pltpu.SemaphoreType.DMA, pltpu.SemaphoreType.DMA]
    ),
)
def _kpool(z_hbm, b_hbm, outs_hbm, outc_hbm, accs, accc, zer_v, zerc_v,
           ones_v, bidx, rows, semi, sems):
    c = lax.axis_index("c")
    s = lax.axis_index("s")
    wid = c * NS + s
    _fill2d(zer_v, 0.0, GRPT)
    _fill(zerc_v, 0.0, 80)
    _fill(ones_v, 1.0, CHUNK)
    base_r = s * GRPT
    pltpu.sync_copy(zer_v, accs.at[pl.ds(base_r, GRPT)])
    pltpu.sync_copy(zerc_v.at[pl.ds(0, GRPT)], accc.at[pl.ds(base_r, GRPT)])
    plsc.subcore_barrier()

    nb0 = wid * (NPAD // NW)

    def body(g, _):
        nb = nb0 + g * CHUNK
        cp1 = pltpu.async_copy(b_hbm.at[pl.ds(nb, CHUNK)], bidx, semi)
        cp2 = pltpu.async_copy(z_hbm.at[pl.ds(nb, CHUNK)], rows, semi)
        cp1.wait()
        cp2.wait()
        pltpu.async_copy(rows, accs.at[bidx], sems, add=True).wait()
        pltpu.async_copy(ones_v, accc.at[bidx], sems, add=True).wait()
        return 0
    lax.fori_loop(0, NCHV, body, 0)
    plsc.subcore_barrier()

    pltpu.sync_copy(accs.at[pl.ds(base_r, GRPT)],
                    outs_hbm.at[c, pl.ds(base_r, GRPT)])
    pltpu.sync_copy(accc.at[pl.ds(base_r, GRPT)],
                    outc_hbm.at[c, pl.ds(base_r, GRPT)])


_HI = lax.Precision.HIGHEST


def kernel(types, pos, edge_index, batch, W1, b1, W2, b2,
           Wg1, bg1, Wg2, bg2, Wg3, bg3, Wo, bo):
    eye8 = jnp.eye(8, dtype=_f32)
    epad = jnp.pad(edge_index.astype(_i32), ((0, 0), (0, EPAD - E)),
                   constant_values=N)

    deg2 = _kdeg(epad)
    deg_pad = deg2[0] + deg2[1] + 1.0
    dinv_pad = lax.rsqrt(deg_pad)
    dinv_big = jnp.matmul(dinv_pad.reshape(NB8, 8),
                          jnp.kron(eye8, jnp.ones((1, H), _f32)),
                          precision=_HI)

    c_cls = types.astype(_i32) * 3 + pos.astype(_i32)
    c_pad = jnp.concatenate([c_cls, jnp.zeros((NPAD - N,), _i32)])
    pk = (lax.bitcast_convert_type(dinv_pad, _i32) & (-16)) | c_pad
    S2 = _kcls(pk, epad)
    S_big = (S2[0] + S2[1]).reshape(NB8, 128)

    tidx = jnp.arange(9) // 3
    pidx = jnp.arange(9) % 3
    ztab = jnp.concatenate(
        [jnp.matmul(jax.nn.one_hot(tidx, 3, dtype=_f32), W1) + b1[None, :],
         jnp.matmul(jax.nn.one_hot(pidx, 3, dtype=_f32), W2) + b2[None, :]],
        axis=1)
    tab16 = jnp.zeros((16, H), _f32).at[:9].set(jnp.matmul(ztab, Wg1))
    z2_big = jax.nn.relu(dinv_big * jnp.matmul(S_big,
                                               jnp.kron(eye8, tab16),
                                               precision=_HI)
                         + jnp.tile(bg1, 8))

    def layer(z_big, W, b, relu_after):
        h_big = jnp.matmul(z_big, jnp.kron(eye8, W))
        g_big = dinv_big * h_big
        T2 = _kmsg(g_big.reshape(NPAD, H), epad).reshape(NC, NB8, 128)
        t_big = T2[0] + T2[1]
        agg = dinv_big * (t_big + g_big) + jnp.tile(b, 8)
        return jax.nn.relu(agg) if relu_after else agg

    z3_big = layer(z2_big, Wg2, bg2, True)
    z4_big = layer(z3_big, Wg3, bg3, False)

    batch_p = jnp.concatenate(
        [batch.astype(_i32),
         G + (jnp.arange(NPAD - N, dtype=_i32) % (GPAD - G))])
    S2p, C2p = _kpool(z4_big.reshape(NPAD, H), batch_p)
    sums = S2p[0, :G] + S2p[1, :G]
    cnt = C2p[0, :G] + C2p[1, :G]
    pooled = sums / jnp.maximum(cnt, 1.0)[:, None]
    return jnp.matmul(pooled, Wo) + bo
